# baseline (speedup 1.0000x reference)
import jax
import jax.numpy as jnp
from jax import lax
from jax.experimental import pallas as pl
from jax.experimental.pallas import tpu as pltpu
from jax.experimental.pallas import tpu_sc as plsc

B, S, D = 4, 4096, 1024
NC, NS = 2, 16
NW = NC * NS
ROWS = B * S
RPW = ROWS // NW
C = 32
G = RPW // C
CW = C * D


def _pos_add_body(emb_hbm, pos_hbm, out_hbm, ebuf, pbuf):
    wid = lax.axis_index("s") * NC + lax.axis_index("c")
    row_base = wid * RPW
    s_base = lax.rem(row_base, S)

    def chunk(g, carry):
        eoff = row_base * D + g * CW
        poff = s_base * D + g * CW
        pltpu.sync_copy(emb_hbm.at[pl.ds(eoff, CW)], ebuf)
        pltpu.sync_copy(pos_hbm.at[pl.ds(poff, CW)], pbuf)

        def addv(i, c2):
            sl = pl.ds(i * 16, 16)
            ebuf[sl] = ebuf[sl] + pbuf[sl]
            return c2

        lax.fori_loop(0, CW // 16, addv, 0)
        pltpu.sync_copy(ebuf, out_hbm.at[pl.ds(eoff, CW)])
        return carry

    lax.fori_loop(0, G, chunk, 0)


@jax.jit
def _run(emb_flat, pos_flat):
    f = pl.kernel(
        _pos_add_body,
        out_type=jax.ShapeDtypeStruct((ROWS * D,), jnp.float32),
        mesh=plsc.VectorSubcoreMesh(
            core_axis_name="c", subcore_axis_name="s",
            num_cores=NC, num_subcores=NS,
        ),
        scratch_types=[
            pltpu.VMEM((CW,), jnp.float32),
            pltpu.VMEM((CW,), jnp.float32),
        ],
    )
    return f(emb_flat, pos_flat)


def kernel(embeddings, pos_table):
    out = _run(embeddings.reshape(-1), pos_table.reshape(-1))
    return out.reshape(B, S, D)


# speedup vs baseline: 1.1539x; 1.1539x over previous
import jax
import jax.numpy as jnp
from jax import lax
from jax.experimental import pallas as pl
from jax.experimental.pallas import tpu as pltpu
from jax.experimental.pallas import tpu_sc as plsc

B, S, D = 4, 4096, 1024
NC, NS = 2, 16
NW = NC * NS
SPW = S // NW
C = 8
G = SPW // C
CW = C * D
NB = 3
NP = 2
NITEMS = G * B


def _pos_add_body(emb_hbm, pos_hbm, out_hbm,
                  eb0, eb1, eb2, ob0, ob1, ob2, pb0, pb1,
                  se0, se1, se2, so0, so1, so2, sp0, sp1):
    ebufs = [eb0, eb1, eb2]
    obufs = [ob0, ob1, ob2]
    pbufs = [pb0, pb1]
    se = [se0, se1, se2]
    so = [so0, so1, so2]
    sp = [sp0, sp1]

    wid = lax.axis_index("s") * NC + lax.axis_index("c")
    seq_base = wid * SPW

    de, dp, do = {}, {}, {}

    def issue_in(i):
        g, b = divmod(i, B)
        k = i % NB
        row = seq_base + (b * S + g * C)
        de[i] = pltpu.async_copy(
            emb_hbm.at[pl.ds(row * D, CW)], ebufs[k], se[k])
        if b == 0:
            prow = seq_base + g * C
            dp[g] = pltpu.async_copy(
                pos_hbm.at[pl.ds(prow * D, CW)], pbufs[g % NP], sp[g % NP])

    for i in range(NB):
        issue_in(i)

    for i in range(NITEMS):
        g, b = divmod(i, B)
        k = i % NB
        de[i].wait()
        if b == 0:
            dp[g].wait()
        if i >= NB:
            do[i - NB].wait()
        eb, ob, pb = ebufs[k], obufs[k], pbufs[g % NP]

        def addv(j, c):
            sl = pl.ds(j * 16, 16)
            ob[sl] = eb[sl] + pb[sl]
            return c

        lax.fori_loop(0, CW // 16, addv, 0, unroll=8)

        row = seq_base + (b * S + g * C)
        do[i] = pltpu.async_copy(
            obufs[k], out_hbm.at[pl.ds(row * D, CW)], so[k])
        if i + NB < NITEMS:
            issue_in(i + NB)

    for i in range(NITEMS - NB, NITEMS):
        do[i].wait()


@jax.jit
def _run(emb_flat, pos_flat):
    f = pl.kernel(
        _pos_add_body,
        out_type=jax.ShapeDtypeStruct((B * S * D,), jnp.float32),
        mesh=plsc.VectorSubcoreMesh(
            core_axis_name="c", subcore_axis_name="s",
            num_cores=NC, num_subcores=NS,
        ),
        scratch_types=(
            [pltpu.VMEM((CW,), jnp.float32)] * (2 * NB + NP)
            + [pltpu.SemaphoreType.DMA] * (2 * NB + NP)
        ),
    )
    return f(emb_flat, pos_flat)


def kernel(embeddings, pos_table):
    out = _run(embeddings.reshape(-1), pos_table.reshape(-1))
    return out.reshape(B, S, D)


# speedup vs baseline: 1.7682x; 1.5324x over previous
import jax
import jax.numpy as jnp
from jax import lax
from jax.experimental import pallas as pl
from jax.experimental.pallas import tpu as pltpu
from jax.experimental.pallas import tpu_sc as plsc

B, S, D = 4, 4096, 1024
NC, NS = 2, 16
NW = NC * NS
SPW = S // NW
C = 8
G = SPW // C
CW = C * D
NB = 3
NP = 2
NITEMS = G * B


def _pos_add_body(emb_hbm, pos_hbm, out_hbm,
                  eb0, eb1, eb2, ob0, ob1, ob2, pb0, pb1,
                  se0, se1, se2, so0, so1, so2, sp0, sp1):
    ebufs = [eb0, eb1, eb2]
    obufs = [ob0, ob1, ob2]
    pbufs = [pb0, pb1]
    se = [se0, se1, se2]
    so = [so0, so1, so2]
    sp = [sp0, sp1]

    wid = lax.axis_index("s") * NC + lax.axis_index("c")
    seq_base = wid * SPW

    de, dp, do = {}, {}, {}

    def issue_in(i):
        g, b = divmod(i, B)
        k = i % NB
        row = seq_base + (b * S + g * C)
        de[i] = pltpu.async_copy(
            emb_hbm.at[pl.ds(row * D, CW)], ebufs[k], se[k])
        if b == 0:
            prow = seq_base + g * C
            dp[g] = pltpu.async_copy(
                pos_hbm.at[pl.ds(prow * D, CW)], pbufs[g % NP], sp[g % NP])

    for i in range(NB):
        issue_in(i)

    for i in range(NITEMS):
        g, b = divmod(i, B)
        k = i % NB
        de[i].wait()
        if b == 0:
            dp[g].wait()
        if i >= NB:
            do[i - NB].wait()
        eb, ob, pb = ebufs[k], obufs[k], pbufs[g % NP]

        @plsc.parallel_loop(0, CW, 16, unroll=8)
        def _(j):
            sl = pl.ds(j, 16)
            ob[sl] = eb[sl] + pb[sl]

        row = seq_base + (b * S + g * C)
        do[i] = pltpu.async_copy(
            obufs[k], out_hbm.at[pl.ds(row * D, CW)], so[k])
        if i + NB < NITEMS:
            issue_in(i + NB)

    for i in range(NITEMS - NB, NITEMS):
        do[i].wait()


@jax.jit
def _run(emb_flat, pos_flat):
    f = pl.kernel(
        _pos_add_body,
        out_type=jax.ShapeDtypeStruct((B * S * D,), jnp.float32),
        mesh=plsc.VectorSubcoreMesh(
            core_axis_name="c", subcore_axis_name="s",
            num_cores=NC, num_subcores=NS,
        ),
        scratch_types=(
            [pltpu.VMEM((CW,), jnp.float32)] * (2 * NB + NP)
            + [pltpu.SemaphoreType.DMA] * (2 * NB + NP)
        ),
    )
    return f(emb_flat, pos_flat)


def kernel(embeddings, pos_table):
    out = _run(embeddings.reshape(-1), pos_table.reshape(-1))
    return out.reshape(B, S, D)


# speedup vs baseline: 4.9464x; 2.7973x over previous
import jax
import jax.numpy as jnp
from jax import lax
from jax.experimental import pallas as pl
from jax.experimental.pallas import tpu as pltpu
from jax.experimental.pallas import tpu_sc as plsc

B, S, D = 4, 4096, 1024
MAXS = 8192
NC, NS = 2, 16
NW = NC * NS
SPW = S // NW
C = 8
G = SPW // C
CW = C * D
NB = 3
NP = 2
NITEMS = G * B


def _pos_add_body(emb_hbm, pos_hbm, out_hbm,
                  eb0, eb1, eb2, ob0, ob1, ob2, pb0, pb1,
                  se0, se1, se2, so0, so1, so2, sp0, sp1):
    ebufs = [eb0, eb1, eb2]
    obufs = [ob0, ob1, ob2]
    pbufs = [pb0, pb1]
    se = [se0, se1, se2]
    so = [so0, so1, so2]
    sp = [sp0, sp1]

    wid = lax.axis_index("s") * NC + lax.axis_index("c")
    seq_base = wid * SPW

    de, dp, do = {}, {}, {}

    def issue_in(i):
        g, b = divmod(i, B)
        k = i % NB
        row = seq_base + g * C
        de[i] = pltpu.async_copy(
            emb_hbm.at[b, pl.ds(row, C), :], ebufs[k], se[k])
        if b == 0:
            dp[g] = pltpu.async_copy(
                pos_hbm.at[pl.ds(row, C), :], pbufs[g % NP], sp[g % NP])

    for i in range(NB):
        issue_in(i)

    for i in range(NITEMS):
        g, b = divmod(i, B)
        k = i % NB
        de[i].wait()
        if b == 0:
            dp[g].wait()
        if i >= NB:
            do[i - NB].wait()
        eb, ob, pb = ebufs[k], obufs[k], pbufs[g % NP]

        def row_add(r, carry):
            @plsc.parallel_loop(0, D, 16, unroll=8)
            def _(c):
                sl = pl.ds(c, 16)
                ob[r, sl] = eb[r, sl] + pb[r, sl]
            return carry

        lax.fori_loop(0, C, row_add, 0)

        row = seq_base + g * C
        do[i] = pltpu.async_copy(
            obufs[k], out_hbm.at[b, pl.ds(row, C), :], so[k])
        if i + NB < NITEMS:
            issue_in(i + NB)

    for i in range(NITEMS - NB, NITEMS):
        do[i].wait()


@jax.jit
def _run(embeddings, pos_table):
    f = pl.kernel(
        _pos_add_body,
        out_type=jax.ShapeDtypeStruct((B, S, D), jnp.float32),
        mesh=plsc.VectorSubcoreMesh(
            core_axis_name="c", subcore_axis_name="s",
            num_cores=NC, num_subcores=NS,
        ),
        scratch_types=(
            [pltpu.VMEM((C, D), jnp.float32)] * (2 * NB + NP)
            + [pltpu.SemaphoreType.DMA] * (2 * NB + NP)
        ),
        compiler_params=pltpu.CompilerParams(use_tc_tiling_on_sc=True),
    )
    return f(embeddings, pos_table)


def kernel(embeddings, pos_table):
    return _run(embeddings, pos_table)


# speedup vs baseline: 5.3073x; 1.0730x over previous
import jax
import jax.numpy as jnp
from jax import lax
from jax.experimental import pallas as pl
from jax.experimental.pallas import tpu as pltpu
from jax.experimental.pallas import tpu_sc as plsc

B, S, D = 4, 4096, 1024
NC, NS = 2, 16
NW = NC * NS
SPW = S // NW
C = 8
G = SPW // C
NB = 3
NP = 2


def _pos_add_body(emb_hbm, pos_hbm, out_hbm, *refs):
    ebufs = [[refs[k * B + b] for b in range(B)] for k in range(NB)]
    pbufs = [refs[NB * B], refs[NB * B + 1]]
    sems = refs[NB * B + NP:]
    se = sems[:NB]
    so = sems[NB:2 * NB]
    sp = sems[2 * NB:2 * NB + NP]

    wid = lax.axis_index("s") * NC + lax.axis_index("c")
    seq_base = wid * SPW

    de, dp, do = {}, {}, {}

    def issue_in(g):
        k = g % NB
        row = seq_base + g * C
        de[g] = [
            pltpu.async_copy(
                emb_hbm.at[b, pl.ds(row, C), :], ebufs[k][b], se[k])
            for b in range(B)
        ]
        dp[g] = pltpu.async_copy(
            pos_hbm.at[pl.ds(row, C), :], pbufs[g % NP], sp[g % NP])

    issue_in(0)

    for g in range(G):
        k = g % NB
        for d in de[g]:
            d.wait()
        dp[g].wait()
        if g >= 2:
            for d in do[g - 2]:
                d.wait()
        if g + 1 < G:
            issue_in(g + 1)
        eb, pb = ebufs[k], pbufs[g % NP]

        def row_add(r, carry):
            @plsc.parallel_loop(0, D, 16, unroll=4)
            def _(c):
                sl = pl.ds(c, 16)
                pv = pb[r, sl]
                for b in range(B):
                    eb[b][r, sl] = eb[b][r, sl] + pv
            return carry

        lax.fori_loop(0, C, row_add, 0)

        row = seq_base + g * C
        do[g] = [
            pltpu.async_copy(
                ebufs[k][b], out_hbm.at[b, pl.ds(row, C), :], so[k])
            for b in range(B)
        ]

    for g in range(max(0, G - 2), G):
        for d in do[g]:
            d.wait()


@jax.jit
def _run(embeddings, pos_table):
    f = pl.kernel(
        _pos_add_body,
        out_type=jax.ShapeDtypeStruct((B, S, D), jnp.float32),
        mesh=plsc.VectorSubcoreMesh(
            core_axis_name="c", subcore_axis_name="s",
            num_cores=NC, num_subcores=NS,
        ),
        scratch_types=(
            [pltpu.VMEM((C, D), jnp.float32)] * (NB * B + NP)
            + [pltpu.SemaphoreType.DMA] * (2 * NB + NP)
        ),
        compiler_params=pltpu.CompilerParams(use_tc_tiling_on_sc=True),
    )
    return f(embeddings, pos_table)


def kernel(embeddings, pos_table):
    return _run(embeddings, pos_table)
